# Initial kernel scaffold; baseline (speedup 1.0000x reference)
import jax
import jax.numpy as jnp
from jax.experimental import pallas as pl


def kernel(x, weight):
    raise NotImplementedError("write your pallas kernel here")


# speedup vs baseline: 1.3100x; 1.3100x over previous
import functools

import jax
import jax.numpy as jnp
from jax import lax
from jax.experimental import pallas as pl
from jax.experimental.pallas import tpu as pltpu
from jax.experimental.pallas import tpu_sc as plsc

D = 32
B = 16384
F = 26
NW = 32
BS = B // NW
PER_W = BS * F

_mesh = plsc.VectorSubcoreMesh(core_axis_name="c", subcore_axis_name="s")


@functools.partial(
    pl.kernel,
    mesh=_mesh,
    out_type=jax.ShapeDtypeStruct((F, 4, 128, 8, 128), jnp.float32),
    scratch_types=[
        pltpu.VMEM((PER_W,), jnp.int32),
        pltpu.VMEM((BS,), jnp.int32),
        pltpu.VMEM((BS,), jnp.int32),
        pltpu.VMEM((BS, 128), jnp.float32),
        pltpu.VMEM((4, 4, 8, 128), jnp.float32),
        pltpu.SemaphoreType.DMA,
    ],
    compiler_params=pltpu.CompilerParams(use_tc_tiling_on_sc=True, needs_layout_passes=False),
)
def _gather_kernel(x_hbm, w_hbm, out_hbm, idx_v, q_v, cb_v, rows_v, blk_v,
                   gsem):
    wid = lax.axis_index("s") * 2 + lax.axis_index("c")
    base = wid * PER_W

    pltpu.sync_copy(x_hbm.at[pl.ds(base, PER_W)], idx_v)

    lanes = lax.iota(jnp.int32, 16)

    def do_field(f, carry):
        def build(g, c2):
            pos = f + F * (16 * g + lanes)
            i = plsc.load_gather(idx_v, [pos])
            q_v[pl.ds(16 * g, 16)] = lax.shift_right_logical(i, 2)
            cb_v[pl.ds(16 * g, 16)] = lax.shift_left(
                lax.bitwise_and(i, 3), 5)
            return c2
        lax.fori_loop(0, BS // 16, build, 0)

        pltpu.async_copy(w_hbm.at[q_v], rows_v, gsem).wait()

        def xpose(g2, c2):
            for c_local in range(4):
                bl = 128 * c_local + 16 * g2 + lanes
                colb = plsc.load_gather(cb_v, [bl])
                for t in range(4):
                    for s in range(8):
                        val = plsc.load_gather(
                            rows_v, [bl, colb + (8 * t + s)])
                        blk_v[t, c_local, s, pl.ds(16 * g2, 16)] = val
            return c2
        lax.fori_loop(0, 8, xpose, 0)

        pltpu.sync_copy(blk_v, out_hbm.at[f, :, pl.ds(4 * wid, 4)])
        return carry

    lax.fori_loop(0, F, do_field, 0)


def kernel(x, weight):
    w128 = weight.reshape(250_000, 128)
    out5 = _gather_kernel(x.reshape(-1), w128)
    return out5.transpose(2, 4, 0, 1, 3).reshape(B, F, D)


# speedup vs baseline: 1.3127x; 1.0021x over previous
import functools

import jax
import jax.numpy as jnp
from jax import lax
from jax.experimental import pallas as pl
from jax.experimental.pallas import tpu as pltpu
from jax.experimental.pallas import tpu_sc as plsc

D = 32
B = 16384
F = 26
NW = 32
BS = B // NW
PER_W = BS * F

_mesh = plsc.VectorSubcoreMesh(core_axis_name="c", subcore_axis_name="s")


@functools.partial(
    pl.kernel,
    mesh=_mesh,
    out_type=jax.ShapeDtypeStruct((F, 4, 128, 8, 128), jnp.float32),
    scratch_types=[
        pltpu.VMEM((PER_W,), jnp.int32),
        pltpu.VMEM((BS,), jnp.int32),
        pltpu.VMEM((BS,), jnp.int32),
        pltpu.VMEM((BS // 2, 128), jnp.float32),
        pltpu.VMEM((BS // 2, 128), jnp.float32),
        pltpu.VMEM((4, 4, 8, 128), jnp.float32),
        pltpu.SemaphoreType.DMA,
        pltpu.SemaphoreType.DMA,
    ],
    compiler_params=pltpu.CompilerParams(use_tc_tiling_on_sc=True, needs_layout_passes=False),
)
def _gather_kernel(x_hbm, w_hbm, out_hbm, idx_v, q_v, cb_v, rows_a, rows_b,
                   blk_v, gsem_a, gsem_b):
    wid = lax.axis_index("s") * 2 + lax.axis_index("c")
    base = wid * PER_W
    half = BS // 2

    pltpu.sync_copy(x_hbm.at[pl.ds(base, PER_W)], idx_v)

    lanes = lax.iota(jnp.int32, 16)
    zeros = lanes * 0

    def do_field(f, carry):
        def build(g, c2):
            pos = f + F * (16 * g + lanes)
            i = plsc.load_gather(idx_v, [pos])
            q_v[pl.ds(16 * g, 16)] = lax.shift_right_logical(i, 2)
            cb_v[pl.ds(16 * g, 16)] = lax.shift_left(
                lax.bitwise_and(i, 3), 5)
            return c2
        lax.fori_loop(0, BS // 16, build, 0)

        cps = []
        for h, (rbuf, sem) in enumerate(((rows_a, gsem_a), (rows_b, gsem_b))):
            cps.append(pltpu.async_copy(
                w_hbm.at[q_v.at[pl.ds(h * half, half)]], rbuf, sem))

        for h, rbuf in enumerate((rows_a, rows_b)):
            cps[h].wait()

            def xpose(g2, c2, h=h, rbuf=rbuf):
                for j in range(2):
                    c_local = 2 * h + j
                    bl = 128 * c_local + 16 * g2 + lanes
                    colb = plsc.load_gather(cb_v, [bl])
                    pbase = lax.shift_left(bl - 128 * (2 * h), 7) + colb
                    for t in range(4):
                        for s in range(8):
                            val = plsc.load_gather(
                                rbuf, [zeros, pbase + (8 * t + s)])
                            blk_v[t, c_local, s, pl.ds(16 * g2, 16)] = val
                return c2
            lax.fori_loop(0, 8, xpose, 0)

        pltpu.sync_copy(blk_v, out_hbm.at[f, :, pl.ds(4 * wid, 4)])
        return carry

    lax.fori_loop(0, F, do_field, 0)


def kernel(x, weight):
    w128 = weight.reshape(250_000, 128)
    out5 = _gather_kernel(x.reshape(-1), w128)
    return out5.transpose(2, 4, 0, 1, 3).reshape(B, F, D)


# speedup vs baseline: 1.8834x; 1.4347x over previous
import functools

import jax
import jax.numpy as jnp
from jax import lax
from jax.experimental import pallas as pl
from jax.experimental.pallas import tpu as pltpu
from jax.experimental.pallas import tpu_sc as plsc

D = 32
B = 16384
F = 26
NW = 32
BS = B // NW
PER_W = BS * F
PITCH = 521

_mesh = plsc.VectorSubcoreMesh(core_axis_name="c", subcore_axis_name="s")


@functools.partial(
    pl.kernel,
    mesh=_mesh,
    out_type=jax.ShapeDtypeStruct((F, 4, 128, 8, 128), jnp.float32),
    scratch_types=[
        pltpu.VMEM((PER_W,), jnp.int32),
        pltpu.VMEM((BS,), jnp.int32),
        pltpu.VMEM((BS,), jnp.int32),
        pltpu.VMEM((BS, D), jnp.float32),
        pltpu.VMEM((BS, D), jnp.float32),
        pltpu.VMEM((D * PITCH,), jnp.float32),
        pltpu.VMEM((4, 4, 8, 128), jnp.float32),
        pltpu.SemaphoreType.DMA,
        pltpu.SemaphoreType.DMA,
    ],
    compiler_params=pltpu.CompilerParams(use_tc_tiling_on_sc=False, needs_layout_passes=False),
)
def _gather_kernel(x_hbm, w_hbm, out_hbm, idx_v, if_a, if_b, rows_a, rows_b,
                   stage_v, blk_v, sem_a, sem_b):
    wid = lax.axis_index("s") * 2 + lax.axis_index("c")
    base = wid * PER_W

    pltpu.sync_copy(x_hbm.at[pl.ds(base, PER_W)], idx_v)

    lanes = lax.iota(jnp.int32, 16)
    dp = [(16 * m + lanes) * PITCH for m in range(2)]

    def build(f, if_v):
        def go(g, c2):
            pos = f + F * (16 * g + lanes)
            if_v[pl.ds(16 * g, 16)] = plsc.load_gather(idx_v, [pos])
            return c2
        lax.fori_loop(0, BS // 16, go, 0)

    def gather(if_v, rows_v, sem):
        return pltpu.async_copy(w_hbm.at[if_v], rows_v, sem)

    def xpose_write(f, rows_v):
        def scat(gb, c2):
            for u in range(4):
                b = 16 * gb + 4 * u
                for v in range(4):
                    bq = b + v
                    for m in range(2):
                        val = plsc.load_gather(
                            rows_v, [bq + lanes * 0, 16 * m + lanes])
                        plsc.store_scatter(stage_v, [dp[m] + bq], val)
            return c2
        lax.fori_loop(0, BS // 16, scat, 0)

        def pack(m, c2):
            for t in range(4):
                for s in range(8):
                    d = 8 * t + s
                    for c in range(4):
                        blk_v[t, c, s, pl.ds(16 * m, 16)] = stage_v[
                            pl.ds(d * PITCH + 128 * c + 16 * m, 16)]
            return c2
        lax.fori_loop(0, 8, pack, 0)

        pltpu.sync_copy(blk_v, out_hbm.at[f, :, pl.ds(4 * wid, 4)])

    build(0, if_a)
    cp_a = gather(if_a, rows_a, sem_a)

    def pair(k, c2):
        f0 = 2 * k
        build(f0 + 1, if_b)
        gather(if_b, rows_b, sem_b)
        pltpu.make_async_copy(w_hbm.at[if_a], rows_a, sem_a).wait()
        xpose_write(f0, rows_a)

        @pl.when(k < F // 2 - 1)
        def _():
            build(f0 + 2, if_a)
            gather(if_a, rows_a, sem_a)

        pltpu.make_async_copy(w_hbm.at[if_b], rows_b, sem_b).wait()
        xpose_write(f0 + 1, rows_b)
        return c2

    lax.fori_loop(0, F // 2, pair, 0)
    _ = cp_a


def kernel(x, weight):
    out5 = _gather_kernel(x.reshape(-1), weight)
    return out5.transpose(2, 4, 0, 1, 3).reshape(B, F, D)


# speedup vs baseline: 1.9241x; 1.0216x over previous
import functools

import jax
import jax.numpy as jnp
from jax import lax
from jax.experimental import pallas as pl
from jax.experimental.pallas import tpu as pltpu
from jax.experimental.pallas import tpu_sc as plsc

D = 32
B = 16384
F = 26
NW = 32
BS = B // NW
PER_W = BS * F
PITCH = 521

_mesh = plsc.VectorSubcoreMesh(core_axis_name="c", subcore_axis_name="s")


@functools.partial(
    pl.kernel,
    mesh=_mesh,
    out_type=jax.ShapeDtypeStruct((F, 4, 128, 8, 128), jnp.float32),
    scratch_types=[
        pltpu.VMEM((PER_W,), jnp.int32),
        pltpu.VMEM((BS,), jnp.int32),
        pltpu.VMEM((BS,), jnp.int32),
        pltpu.VMEM((BS, D), jnp.float32),
        pltpu.VMEM((BS, D), jnp.float32),
        pltpu.VMEM((D * PITCH,), jnp.float32),
        pltpu.VMEM((4, 4, 8, 128), jnp.float32),
        pltpu.VMEM((4, 4, 8, 128), jnp.float32),
        pltpu.SemaphoreType.DMA,
        pltpu.SemaphoreType.DMA,
        pltpu.SemaphoreType.DMA,
        pltpu.SemaphoreType.DMA,
    ],
    compiler_params=pltpu.CompilerParams(use_tc_tiling_on_sc=False, needs_layout_passes=False),
)
def _gather_kernel(x_hbm, w_hbm, out_hbm, idx_v, if_a, if_b, rows_a, rows_b,
                   stage_v, blk_a, blk_b, sem_a, sem_b, wsem_a, wsem_b):
    wid = lax.axis_index("s") * 2 + lax.axis_index("c")
    base = wid * PER_W

    pltpu.sync_copy(x_hbm.at[pl.ds(base, PER_W)], idx_v)

    lanes = lax.iota(jnp.int32, 16)
    dp = [(16 * m + lanes) * PITCH for m in range(2)]

    def build(f, if_v):
        def go(g, c2):
            pos = f + F * (16 * g + lanes)
            if_v[pl.ds(16 * g, 16)] = plsc.load_gather(idx_v, [pos])
            return c2
        lax.fori_loop(0, BS // 16, go, 0)

    def gather(if_v, rows_v, sem):
        return pltpu.async_copy(w_hbm.at[if_v], rows_v, sem)

    def xpose_write(f, rows_v, blk_v, wsem):
        def scat(gb, c2):
            for u in range(4):
                b = 16 * gb + 4 * u
                for v in range(4):
                    bq = b + v
                    for m in range(2):
                        val = plsc.load_gather(
                            rows_v, [bq + lanes * 0, 16 * m + lanes])
                        plsc.store_scatter(stage_v, [dp[m] + bq], val)
            return c2
        lax.fori_loop(0, BS // 16, scat, 0)

        pltpu.make_async_copy(
            out_hbm.at[0, :, pl.ds(4 * wid, 4)], blk_v, wsem).wait()

        def pack(m, c2):
            for t in range(4):
                for s in range(8):
                    d = 8 * t + s
                    for c in range(4):
                        blk_v[t, c, s, pl.ds(16 * m, 16)] = stage_v[
                            pl.ds(d * PITCH + 128 * c + 16 * m, 16)]
            return c2
        lax.fori_loop(0, 8, pack, 0)

        pltpu.async_copy(blk_v, out_hbm.at[f, :, pl.ds(4 * wid, 4)], wsem)

    pltpu.async_copy(blk_a, out_hbm.at[0, :, pl.ds(4 * wid, 4)], wsem_a)
    pltpu.async_copy(blk_b, out_hbm.at[1, :, pl.ds(4 * wid, 4)], wsem_b)
    build(0, if_a)
    cp_a = gather(if_a, rows_a, sem_a)

    def pair(k, c2):
        f0 = 2 * k
        build(f0 + 1, if_b)
        gather(if_b, rows_b, sem_b)
        pltpu.make_async_copy(w_hbm.at[if_a], rows_a, sem_a).wait()
        xpose_write(f0, rows_a, blk_a, wsem_a)

        @pl.when(k < F // 2 - 1)
        def _():
            build(f0 + 2, if_a)
            gather(if_a, rows_a, sem_a)

        pltpu.make_async_copy(w_hbm.at[if_b], rows_b, sem_b).wait()
        xpose_write(f0 + 1, rows_b, blk_b, wsem_b)
        return c2

    lax.fori_loop(0, F // 2, pair, 0)
    _ = cp_a

    pltpu.make_async_copy(out_hbm.at[0, :, pl.ds(4 * wid, 4)], blk_a,
                          wsem_a).wait()
    pltpu.make_async_copy(out_hbm.at[0, :, pl.ds(4 * wid, 4)], blk_b,
                          wsem_b).wait()


def kernel(x, weight):
    out5 = _gather_kernel(x.reshape(-1), weight)
    return out5.transpose(2, 4, 0, 1, 3).reshape(B, F, D)
